# baseline (speedup 1.0000x reference)
import functools

import jax
import jax.numpy as jnp
import numpy as np
from jax import lax
from jax.experimental import pallas as pl
from jax.experimental.pallas import tpu as pltpu
from jax.experimental.pallas import tpu_sc as plsc

SEQ_LENGTH = 8192
EMBEDDING_DIM = 64
NUM_CORES = 2
NUM_SUBCORES = 16
NUM_WORKERS = NUM_CORES * NUM_SUBCORES
ROWS_PER_WORKER = SEQ_LENGTH // NUM_WORKERS
GATHER_CHUNK = 128
CHUNKS_PER_WORKER = ROWS_PER_WORKER // GATHER_CHUNK


def _positional_encoding():
    pos = np.arange(SEQ_LENGTH, dtype=np.float64)[:, None]
    pe = np.zeros((SEQ_LENGTH, EMBEDDING_DIM), dtype=np.float64)
    i_even = np.arange(0, EMBEDDING_DIM, 2)
    i_odd = i_even + 1
    pe[:, i_even] = np.sin(pos / 10000 ** (2.0 * i_even / EMBEDDING_DIM))
    pe[:, i_odd] = np.cos(pos / 10000 ** (2.0 * i_odd / EMBEDDING_DIM))
    return pe.astype(np.float32)


_PE = _positional_encoding().reshape(NUM_WORKERS, ROWS_PER_WORKER, EMBEDDING_DIM)


def _sc_embed(table, idx3, pe3):
    mesh = plsc.VectorSubcoreMesh(core_axis_name="c", subcore_axis_name="s")

    @functools.partial(
        pl.kernel,
        mesh=mesh,
        compiler_params=pltpu.CompilerParams(use_tc_tiling_on_sc=False),
        out_type=jax.ShapeDtypeStruct((SEQ_LENGTH, EMBEDDING_DIM), jnp.float32),
        scratch_types=[
            pltpu.VMEM((CHUNKS_PER_WORKER, GATHER_CHUNK), jnp.int32),
            pltpu.VMEM((ROWS_PER_WORKER, EMBEDDING_DIM), jnp.float32),
            pltpu.VMEM((ROWS_PER_WORKER, EMBEDDING_DIM), jnp.float32),
            pltpu.SemaphoreType.DMA,
            pltpu.SemaphoreType.DMA,
        ],
    )
    def k(table_hbm, idx_hbm, pe_hbm, out_hbm, idx_v, rows_v, pe_v, gsem, psem):
        wid = lax.axis_index("s") * NUM_CORES + lax.axis_index("c")
        base = wid * ROWS_PER_WORKER

        pe_cp = pltpu.async_copy(pe_hbm.at[wid], pe_v, psem)
        pltpu.sync_copy(idx_hbm.at[wid], idx_v)
        gathers = []
        for j in range(CHUNKS_PER_WORKER):
            gathers.append(
                pltpu.async_copy(
                    table_hbm.at[idx_v.at[j]],
                    rows_v.at[pl.ds(j * GATHER_CHUNK, GATHER_CHUNK)],
                    gsem,
                )
            )
        for g in gathers:
            g.wait()
        pe_cp.wait()

        @pl.loop(0, ROWS_PER_WORKER)
        def _(r):
            for c in range(EMBEDDING_DIM // 16):
                sl = pl.ds(c * 16, 16)
                plsc.addupdate(rows_v.at[r, sl], pe_v[r, sl])

        pltpu.sync_copy(rows_v, out_hbm.at[pl.ds(base, ROWS_PER_WORKER)])

    return k(table, idx3, pe3)


def kernel(input_indices, table):
    idx3 = input_indices.astype(jnp.int32).reshape(
        NUM_WORKERS, CHUNKS_PER_WORKER, GATHER_CHUNK
    )
    pe3 = jnp.asarray(_PE)
    return _sc_embed(table, idx3, pe3)
